# baseline (speedup 1.0000x reference)
import functools

import jax
import jax.numpy as jnp
from jax import lax
from jax.experimental import pallas as pl
from jax.experimental.pallas import tpu as pltpu
from jax.experimental.pallas import tpu_sc as plsc

D_MODEL = 64
SCALE = 8.0
LANES = 16
ROWS_PER_GATHER = 128
CHUNK_IDX_ROWS = 4


@functools.lru_cache(maxsize=None)
def _build(B):
    info = plsc.get_sparse_core_info()
    NC, NS = info.num_cores, info.num_subcores
    NW = NC * NS
    b_per_w = B // NW
    idx_rows_per_w = b_per_w // ROWS_PER_GATHER
    chunks = idx_rows_per_w // CHUNK_IDX_ROWS
    C = CHUNK_IDX_ROWS * ROWS_PER_GATHER
    mesh = plsc.VectorSubcoreMesh(core_axis_name="c", subcore_axis_name="s")

    @functools.partial(
        pl.kernel,
        mesh=mesh,
        out_type=jax.ShapeDtypeStruct((B, D_MODEL), jnp.float32),
        scratch_types=[
            pltpu.VMEM((CHUNK_IDX_ROWS, ROWS_PER_GATHER), jnp.int32),
            pltpu.VMEM((C, D_MODEL), jnp.float32),
            pltpu.SemaphoreType.DMA,
        ],
        compiler_params=pltpu.CompilerParams(use_tc_tiling_on_sc=False),
    )
    def k(idx_hbm, table_hbm, out_hbm, idx_v, rows_v, sem):
        wid = lax.axis_index("s") * NC + lax.axis_index("c")
        row0 = wid * idx_rows_per_w

        def chunk_body(ci, carry):
            irow = row0 + ci * CHUNK_IDX_ROWS
            pltpu.sync_copy(idx_hbm.at[pl.ds(irow, CHUNK_IDX_ROWS)], idx_v)
            copies = [
                pltpu.async_copy(
                    table_hbm.at[idx_v.at[j]],
                    rows_v.at[pl.ds(j * ROWS_PER_GATHER, ROWS_PER_GATHER)],
                    sem,
                )
                for j in range(CHUNK_IDX_ROWS)
            ]
            for cp in copies:
                cp.wait()

            def scale_body(i, c2):
                for j in range(D_MODEL // LANES):
                    sl = pl.ds(j * LANES, LANES)
                    rows_v[i, sl] = rows_v[i, sl] * SCALE
                return c2

            lax.fori_loop(0, C, scale_body, 0, unroll=4)
            pltpu.sync_copy(rows_v, out_hbm.at[pl.ds(irow * ROWS_PER_GATHER, C)])
            return carry

        lax.fori_loop(0, chunks, chunk_body, 0)

    return k


def kernel(x, table):
    B = x.shape[0] * x.shape[1]
    idx = x.reshape(-1).astype(jnp.int32).reshape(B // ROWS_PER_GATHER,
                                                  ROWS_PER_GATHER)
    out = _build(B)(idx, table)
    return out.reshape(x.shape[0], x.shape[1], D_MODEL)


# speedup vs baseline: 1.0552x; 1.0552x over previous
import functools

import jax
import jax.numpy as jnp
from jax import lax
from jax.experimental import pallas as pl
from jax.experimental.pallas import tpu as pltpu
from jax.experimental.pallas import tpu_sc as plsc

D_MODEL = 64
SCALE = 8.0
LANES = 16
ROWS_PER_GATHER = 128
GATHERS_PER_CHUNK = 5
C = GATHERS_PER_CHUNK * ROWS_PER_GATHER
NBUF = 2


@functools.lru_cache(maxsize=None)
def _build(B):
    info = plsc.get_sparse_core_info()
    NC, NS = info.num_cores, info.num_subcores
    NW = NC * NS
    b_per_w = B // NW
    idx_rows_w = b_per_w // ROWS_PER_GATHER
    chunks = b_per_w // C
    mesh = plsc.VectorSubcoreMesh(core_axis_name="c", subcore_axis_name="s")

    @functools.partial(
        pl.kernel,
        mesh=mesh,
        out_type=jax.ShapeDtypeStruct((B, D_MODEL), jnp.float32),
        scratch_types=[
            pltpu.VMEM((idx_rows_w, ROWS_PER_GATHER), jnp.int32),
            pltpu.VMEM((NBUF, C, D_MODEL), jnp.float32),
            pltpu.SemaphoreType.DMA,
            pltpu.SemaphoreType.DMA,
        ],
        compiler_params=pltpu.CompilerParams(use_tc_tiling_on_sc=False),
    )
    def k(idx_hbm, table_hbm, out_hbm, idx_v, rows_v, gsem, wsem):
        wid = lax.axis_index("s") * NC + lax.axis_index("c")
        row0 = wid * idx_rows_w
        out0 = wid * b_per_w
        pltpu.sync_copy(idx_hbm.at[pl.ds(row0, idx_rows_w)], idx_v)

        def fire_gather(g, b):
            for j in range(GATHERS_PER_CHUNK):
                pltpu.async_copy(
                    table_hbm.at[idx_v.at[g * GATHERS_PER_CHUNK + j]],
                    rows_v.at[b, pl.ds(j * ROWS_PER_GATHER, ROWS_PER_GATHER)],
                    gsem,
                )

        def drain_one_chunk(sem, b):
            pltpu.make_async_copy(
                table_hbm.at[pl.ds(0, C)], rows_v.at[b], sem
            ).wait()

        fire_gather(0, 0)

        def outer(go, carry):
            for b in range(NBUF):
                g = go * NBUF + b
                drain_one_chunk(gsem, b)

                def scale_body(i, c2):
                    for j in range(D_MODEL // LANES):
                        sl = pl.ds(j * LANES, LANES)
                        rows_v[b, i, sl] = rows_v[b, i, sl] * SCALE
                    return c2

                lax.fori_loop(0, C, scale_body, 0, unroll=8)

                nb = (b + 1) % NBUF

                @pl.when(g >= 1)
                def _():
                    drain_one_chunk(wsem, nb)

                @pl.when(g + 1 < chunks)
                def _():
                    fire_gather(g + 1, nb)

                pltpu.async_copy(
                    rows_v.at[b], out_hbm.at[pl.ds(out0 + g * C, C)], wsem
                )
            return carry

        lax.fori_loop(0, chunks // NBUF, outer, 0)
        drain_one_chunk(wsem, (chunks - 1) % NBUF)

    return k


def kernel(x, table):
    B = x.shape[0] * x.shape[1]
    idx = x.reshape(-1).astype(jnp.int32).reshape(B // ROWS_PER_GATHER,
                                                  ROWS_PER_GATHER)
    out = _build(B)(idx, table)
    return out.reshape(x.shape[0], x.shape[1], D_MODEL)
